# baseline (speedup 1.0000x reference)
import functools

import jax
import jax.numpy as jnp
from jax import lax
from jax.experimental import pallas as pl
from jax.experimental.pallas import tpu as pltpu
from jax.experimental.pallas import tpu_sc as plsc

Q, K, D = 128, 2048, 256
KNN = 6
NEG_INF = float("-inf")
M = 16


def _exact_tc_kernel(micro_ref, micro_all_ref, w_ref, idx_out_ref):
    micro = micro_ref[...]
    micro_all = micro_all_ref[...]
    v = w_ref[...] - jnp.float32(1.0 / D)

    xs = micro * v
    cross = lax.dot_general(
        xs, micro_all,
        dimension_numbers=(((1,), (1,)), ((), ())),
        preferred_element_type=jnp.float32,
        precision=lax.Precision.HIGHEST)
    maa = micro_all * micro_all
    rowa = lax.dot_general(
        v, maa,
        dimension_numbers=(((1,), (1,)), ((), ())),
        preferred_element_type=jnp.float32,
        precision=lax.Precision.HIGHEST)
    work = rowa - 2.0 * cross

    iota_k = lax.broadcasted_iota(jnp.int32, (Q, K), 1)
    idxs = []
    for _ in range(M):
        m = jnp.max(work, axis=1, keepdims=True)
        idx_j = jnp.max(jnp.where(work == m, iota_k, -1),
                        axis=1, keepdims=True)
        work = jnp.where(iota_k == idx_j, NEG_INF, work)
        idxs.append(idx_j)
    idx_out_ref[...] = jnp.concatenate(idxs, axis=1)


def _run_exact_tc(micro, micro_all, W):
    return pl.pallas_call(
        _exact_tc_kernel,
        out_shape=jax.ShapeDtypeStruct((Q, M), jnp.int32),
    )(micro, micro_all, W)


_NC, _NS = 2, 16
_NW = _NC * _NS
_QPW = Q // _NW
_DC = D // 16


def _round_bf16(x):
    xi = lax.bitcast_convert_type(x, jnp.int32)
    rounded = (xi + 0x7FFF + ((xi >> 16) & 1)) & jnp.int32(-65536)
    return lax.bitcast_convert_type(rounded, jnp.float32)


def _sc_kernel(micro_hbm, table_hbm, w_hbm, idx16_hbm,
               lab_hbm, laball_hbm, out_hbm, closs_hbm,
               mic_v, w_v, wr_v, lab_v, laball_v,
               idx_v, rows_v, out_v, cl_v, sem):
    wid = lax.axis_index("s") * _NC + lax.axis_index("c")
    qbase = wid * _QPW
    lane = lax.iota(jnp.int32, 16)

    pltpu.sync_copy(micro_hbm.at[pl.ds(qbase, _QPW)], mic_v)
    pltpu.sync_copy(w_hbm.at[pl.ds(0, 1)], w_v)
    pltpu.sync_copy(idx16_hbm.at[pl.ds(qbase, _QPW)], idx_v)
    pltpu.sync_copy(lab_hbm.at[pl.ds(qbase, _QPW)], lab_v)
    pltpu.sync_copy(laball_hbm, laball_v)

    for ck in range(_DC):
        wr_v[0, pl.ds(ck * 16, 16)] = _round_bf16(w_v[0, pl.ds(ck * 16, 16)])

    closs_part = jnp.zeros((16,), jnp.float32)

    for r in range(_QPW):
        pltpu.async_copy(table_hbm.at[idx_v.at[r]], rows_v, sem).wait()
        kv = idx_v[r, :]

        def cand_body(c, v0c):
            accs = [jnp.zeros((16,), jnp.float32) for _ in range(2)]
            accms = [jnp.zeros((16,), jnp.float32) for _ in range(2)]
            for ck in range(_DC):
                a = rows_v[c, pl.ds(ck * 16, 16)]
                mm = mic_v[r, pl.ds(ck * 16, 16)]
                wrv = wr_v[0, pl.ds(ck * 16, 16)]
                diff = a - mm
                dis = diff * diff
                accms[ck % 2] = accms[ck % 2] + dis
                accs[ck % 2] = accs[ck % 2] + _round_bf16(dis) * wrv
            noisy_s = (jnp.sum(accs[0] + accs[1], axis=0)
                       - jnp.sum(accms[0] + accms[1], axis=0)
                       * jnp.float32(1.0 / D))
            nv = jnp.full((16,), noisy_s, jnp.float32)
            return jnp.where(lane == jnp.full((16,), c, jnp.int32), nv, v0c)

        v0 = lax.fori_loop(0, M, cand_body,
                           jnp.full((16,), NEG_INF, jnp.float32))

        sel_vals = jnp.full((16,), NEG_INF, jnp.float32)
        sel_pos = jnp.zeros((16,), jnp.int32)
        sel_k = jnp.zeros((16,), jnp.int32)
        for j in range(KNN):
            mj = jnp.max(v0, axis=0)
            mjv = jnp.full((16,), mj, jnp.float32)
            kj = jnp.max(jnp.where(v0 == mjv, kv, -1), axis=0)
            kjv = jnp.full((16,), kj, jnp.int32)
            hit = (v0 == mjv) & (kv == kjv)
            pj = jnp.max(jnp.where(hit, lane, -1), axis=0)
            v0 = jnp.where(hit, NEG_INF, v0)
            is_j = lane == j
            sel_vals = jnp.where(is_j, mjv, sel_vals)
            sel_pos = jnp.where(is_j, jnp.full((16,), pj, jnp.int32), sel_pos)
            sel_k = jnp.where(is_j, kjv, sel_k)

        mx = jnp.max(jnp.where(lane < KNN, sel_vals, NEG_INF), axis=0)
        ez = jnp.exp(sel_vals - jnp.full((16,), mx, jnp.float32))
        ez = jnp.where(lane < KNN, ez, 0.0)
        denom = jnp.sum(ez, axis=0)
        wvec = ez / jnp.full((16,), denom, jnp.float32)

        wq = [jnp.sum(jnp.where(lane == j, wvec, 0.0), axis=0)
              for j in range(KNN)]
        pq = [jnp.sum(jnp.where(lane == j, sel_pos, 0), axis=0)
              for j in range(KNN)]
        wr = [_round_bf16(jnp.full((16,), wq[j], jnp.float32))
              for j in range(KNN)]
        for c in range(_DC):
            col = lane + c * 16
            acc = mic_v[r, pl.ds(c * 16, 16)]
            for j in range(KNN):
                a = plsc.load_gather(
                    rows_v, [jnp.full((16,), pq[j], jnp.int32), col])
                acc = acc + wr[j] * _round_bf16(a)
            out_v[r, pl.ds(c * 16, 16)] = acc

        la = plsc.load_gather(laball_v, [sel_k])
        dl = jnp.abs(la - lab_v[r, :])
        closs_part = closs_part + jnp.where(lane < KNN, wvec * dl, 0.0)

    cl_v[0, :] = closs_part
    pltpu.sync_copy(out_v, out_hbm.at[pl.ds(qbase, _QPW)])
    pltpu.sync_copy(cl_v, closs_hbm.at[pl.ds(wid, 1)])


@functools.cache
def _build_sc():
    return functools.partial(
        pl.kernel,
        mesh=plsc.VectorSubcoreMesh(core_axis_name="c", subcore_axis_name="s"),
        compiler_params=pltpu.CompilerParams(needs_layout_passes=False),
        out_type=(
            jax.ShapeDtypeStruct((Q, D), jnp.float32),
            jax.ShapeDtypeStruct((_NW, 16), jnp.float32),
        ),
        scratch_types=[
            pltpu.VMEM((_QPW, D), jnp.float32),
            pltpu.VMEM((1, D), jnp.float32),
            pltpu.VMEM((1, D), jnp.float32),
            pltpu.VMEM((_QPW, 16), jnp.float32),
            pltpu.VMEM((K,), jnp.float32),
            pltpu.VMEM((_QPW, M), jnp.int32),
            pltpu.VMEM((M, D), jnp.float32),
            pltpu.VMEM((_QPW, D), jnp.float32),
            pltpu.VMEM((1, 16), jnp.float32),
            pltpu.SemaphoreType.DMA,
        ],
    )(_sc_kernel)


def kernel(micro, label, micro_all, label_all, W):
    idx16 = _run_exact_tc(micro, micro_all, W)
    lab_b = jnp.broadcast_to(label[:, None], (Q, 16))
    micro_tmp, cl_parts = _build_sc()(
        micro, micro_all, W, idx16, lab_b, label_all)
    co_loss = jnp.float32(1e-4) + jnp.sum(cl_parts[:, :KNN]) / Q
    return micro_tmp, co_loss


# speedup vs baseline: 1.0022x; 1.0022x over previous
import functools

import jax
import jax.numpy as jnp
from jax import lax
from jax.experimental import pallas as pl
from jax.experimental.pallas import tpu as pltpu
from jax.experimental.pallas import tpu_sc as plsc

Q, K, D = 128, 2048, 256
KNN = 6
NEG_INF = float("-inf")
M = 16


def _exact_tc_kernel(micro_ref, micro_all_ref, w_ref, idx_out_ref):
    micro = micro_ref[...]
    micro_all = micro_all_ref[...]
    v = w_ref[...] - jnp.float32(1.0 / D)

    xs = micro * v
    cross = lax.dot_general(
        xs, micro_all,
        dimension_numbers=(((1,), (1,)), ((), ())),
        preferred_element_type=jnp.float32,
        precision=lax.Precision.HIGHEST)
    maa = micro_all * micro_all
    rowa = lax.dot_general(
        v, maa,
        dimension_numbers=(((1,), (1,)), ((), ())),
        preferred_element_type=jnp.float32,
        precision=lax.Precision.HIGHEST)
    work = rowa - 2.0 * cross

    iota_k = lax.broadcasted_iota(jnp.int32, (Q, K), 1)
    idxs = []
    for _ in range(M):
        m = jnp.max(work, axis=1, keepdims=True)
        idx_j = jnp.max(jnp.where(work == m, iota_k, -1),
                        axis=1, keepdims=True)
        work = jnp.where(iota_k == idx_j, NEG_INF, work)
        idxs.append(idx_j)
    idx_out_ref[...] = jnp.concatenate(idxs, axis=1)


def _run_exact_tc(micro, micro_all, W):
    return pl.pallas_call(
        _exact_tc_kernel,
        out_shape=jax.ShapeDtypeStruct((Q, M), jnp.int32),
    )(micro, micro_all, W)


_NC, _NS = 2, 16
_NW = _NC * _NS
_QPW = Q // _NW
_DC = D // 16


def _round_bf16(x):
    xi = lax.bitcast_convert_type(x, jnp.int32)
    rounded = (xi + 0x7FFF + ((xi >> 16) & 1)) & jnp.int32(-65536)
    return lax.bitcast_convert_type(rounded, jnp.float32)


def _sc_kernel(micro_hbm, table_hbm, w_hbm, idx16_hbm,
               lab_hbm, laball_hbm, out_hbm, closs_hbm,
               mic_v, w_v, wr_v, lab_v, laball_v,
               idx_v, rows_v, out_v, cl_v, sem):
    wid = lax.axis_index("s") * _NC + lax.axis_index("c")
    qbase = wid * _QPW
    lane = lax.iota(jnp.int32, 16)

    pltpu.sync_copy(micro_hbm.at[pl.ds(qbase, _QPW)], mic_v)
    pltpu.sync_copy(w_hbm.at[pl.ds(0, 1)], w_v)
    pltpu.sync_copy(idx16_hbm.at[pl.ds(qbase, _QPW)], idx_v)
    pltpu.sync_copy(lab_hbm.at[pl.ds(qbase, _QPW)], lab_v)
    pltpu.sync_copy(laball_hbm, laball_v)

    for ck in range(_DC):
        wr_v[0, pl.ds(ck * 16, 16)] = _round_bf16(w_v[0, pl.ds(ck * 16, 16)])

    closs_part = jnp.zeros((16,), jnp.float32)

    for r in range(_QPW):
        pltpu.async_copy(table_hbm.at[idx_v.at[r]], rows_v, sem).wait()
        kv = idx_v[r, :]

        def cand_body(c, v0c):
            acc = jnp.zeros((16,), jnp.float32)
            accm = jnp.zeros((16,), jnp.float32)
            for ck in range(_DC):
                a = rows_v[c, pl.ds(ck * 16, 16)]
                mm = mic_v[r, pl.ds(ck * 16, 16)]
                wrv = wr_v[0, pl.ds(ck * 16, 16)]
                diff = a - mm
                dis = diff * diff
                accm = accm + dis
                acc = acc + _round_bf16(dis) * wrv
            noisy_s = (jnp.sum(acc, axis=0)
                       - jnp.sum(accm, axis=0) * jnp.float32(1.0 / D))
            nv = jnp.full((16,), noisy_s, jnp.float32)
            return jnp.where(lane == jnp.full((16,), c, jnp.int32), nv, v0c)

        v0 = lax.fori_loop(0, M, cand_body,
                           jnp.full((16,), NEG_INF, jnp.float32))

        sel_vals = jnp.full((16,), NEG_INF, jnp.float32)
        sel_pos = jnp.zeros((16,), jnp.int32)
        sel_k = jnp.zeros((16,), jnp.int32)
        for j in range(KNN):
            mj = jnp.max(v0, axis=0)
            mjv = jnp.full((16,), mj, jnp.float32)
            kj = jnp.max(jnp.where(v0 == mjv, kv, -1), axis=0)
            kjv = jnp.full((16,), kj, jnp.int32)
            hit = (v0 == mjv) & (kv == kjv)
            pj = jnp.max(jnp.where(hit, lane, -1), axis=0)
            v0 = jnp.where(hit, NEG_INF, v0)
            is_j = lane == j
            sel_vals = jnp.where(is_j, mjv, sel_vals)
            sel_pos = jnp.where(is_j, jnp.full((16,), pj, jnp.int32), sel_pos)
            sel_k = jnp.where(is_j, kjv, sel_k)

        mx = jnp.max(jnp.where(lane < KNN, sel_vals, NEG_INF), axis=0)
        ez = jnp.exp(sel_vals - jnp.full((16,), mx, jnp.float32))
        ez = jnp.where(lane < KNN, ez, 0.0)
        denom = jnp.sum(ez, axis=0)
        wvec = ez / jnp.full((16,), denom, jnp.float32)

        wq = [jnp.sum(jnp.where(lane == j, wvec, 0.0), axis=0)
              for j in range(KNN)]
        pq = [jnp.sum(jnp.where(lane == j, sel_pos, 0), axis=0)
              for j in range(KNN)]
        wr = [_round_bf16(jnp.full((16,), wq[j], jnp.float32))
              for j in range(KNN)]
        for c in range(_DC):
            col = lane + c * 16
            acc = mic_v[r, pl.ds(c * 16, 16)]
            for j in range(KNN):
                a = plsc.load_gather(
                    rows_v, [jnp.full((16,), pq[j], jnp.int32), col])
                acc = acc + wr[j] * _round_bf16(a)
            out_v[r, pl.ds(c * 16, 16)] = acc

        la = plsc.load_gather(laball_v, [sel_k])
        dl = jnp.abs(la - lab_v[r, :])
        closs_part = closs_part + jnp.where(lane < KNN, wvec * dl, 0.0)

    cl_v[0, :] = closs_part
    pltpu.sync_copy(out_v, out_hbm.at[pl.ds(qbase, _QPW)])
    pltpu.sync_copy(cl_v, closs_hbm.at[pl.ds(wid, 1)])


@functools.cache
def _build_sc():
    return functools.partial(
        pl.kernel,
        mesh=plsc.VectorSubcoreMesh(core_axis_name="c", subcore_axis_name="s"),
        compiler_params=pltpu.CompilerParams(needs_layout_passes=False),
        out_type=(
            jax.ShapeDtypeStruct((Q, D), jnp.float32),
            jax.ShapeDtypeStruct((_NW, 16), jnp.float32),
        ),
        scratch_types=[
            pltpu.VMEM((_QPW, D), jnp.float32),
            pltpu.VMEM((1, D), jnp.float32),
            pltpu.VMEM((1, D), jnp.float32),
            pltpu.VMEM((_QPW, 16), jnp.float32),
            pltpu.VMEM((K,), jnp.float32),
            pltpu.VMEM((_QPW, M), jnp.int32),
            pltpu.VMEM((M, D), jnp.float32),
            pltpu.VMEM((_QPW, D), jnp.float32),
            pltpu.VMEM((1, 16), jnp.float32),
            pltpu.SemaphoreType.DMA,
        ],
    )(_sc_kernel)


def kernel(micro, label, micro_all, label_all, W):
    idx16 = _run_exact_tc(micro, micro_all, W)
    lab_b = jnp.broadcast_to(label[:, None], (Q, 16))
    micro_tmp, cl_parts = _build_sc()(
        micro, micro_all, W, idx16, lab_b, label_all)
    co_loss = jnp.float32(1e-4) + jnp.sum(cl_parts[:, :KNN]) / Q
    return micro_tmp, co_loss
